# baseline (speedup 1.0000x reference)
import functools

import jax
import jax.numpy as jnp
import numpy as np
from jax import lax
from jax.experimental import pallas as pl
from jax.experimental.pallas import tpu as pltpu
from jax.experimental.pallas import tpu_sc as plsc

N_SCEN_SEL = 32

_NC = 2
_NS = 16
_NW = _NC * _NS
_L = 16

_PERM_CACHE = {}
try:
    with jax.default_device(jax.devices("cpu")[0]):
        _PERM_CACHE[512] = np.asarray(
            jax.random.permutation(jax.random.key(42), 512)
        )
except Exception:
    pass

_REP = 4


def _sc_one_hot(idx, b, s):
    k = idx.shape[0]
    assert k == _NW, "one worker per selected scenario"
    rpw = b // _NW
    assert rpw % _REP == 0 and s % _L == 0

    def body(idx_hbm, p_hbm, idx_v, pat_v, psem):
        wid = lax.axis_index("s") * _NC + lax.axis_index("c")
        pltpu.sync_copy(idx_hbm, idx_v)
        io = lax.iota(jnp.int32, _L)
        v_lo = idx_v[pl.ds(0, _L)]
        v_hi = idx_v[pl.ds(_L, _L)]

        z16 = jnp.zeros((_L,), jnp.float32)
        cch = s // _L

        def zero_body(kk, _):
            for r in range(_REP):
                for c in range(cch):
                    pat_v[r, kk, pl.ds(c * _L, _L)] = z16
            return 0

        lax.fori_loop(0, k, zero_body, 0)
        ones = jnp.full((_L,), 1.0, dtype=jnp.float32)
        for r in range(_REP):
            rv = jnp.full((_L,), r, dtype=jnp.int32)
            plsc.store_scatter(pat_v, [rv, io, v_lo], ones)
            plsc.store_scatter(pat_v, [rv, io + _L, v_hi], ones)

        pcopies = [
            pltpu.async_copy(
                pat_v, p_hbm.at[pl.ds(wid * rpw + i * _REP, _REP)], psem)
            for i in range(rpw // _REP)
        ]
        for c in pcopies:
            c.wait()

    mesh = plsc.VectorSubcoreMesh(core_axis_name="c", subcore_axis_name="s")
    f = pl.kernel(
        body,
        out_type=jax.ShapeDtypeStruct((b, k, s), jnp.float32),
        mesh=mesh,
        scratch_types=[
            pltpu.VMEM((k,), jnp.int32),
            pltpu.VMEM((_REP, k, s), jnp.float32),
            pltpu.SemaphoreType.DMA,
        ],
        compiler_params=pltpu.CompilerParams(needs_layout_passes=False),
    )
    return f(idx)


def _tc_gather(y3, idx, k, t, b):
    def body(idx_ref, y_ref, out_ref):
        out_ref[...] = y_ref[...]

    grid_spec = pltpu.PrefetchScalarGridSpec(
        num_scalar_prefetch=1,
        grid=(k,),
        in_specs=[
            pl.BlockSpec((1, t, b), lambda i, idx_ref: (idx_ref[i], 0, 0))
        ],
        out_specs=pl.BlockSpec((1, t, b), lambda i, idx_ref: (i, 0, 0)),
    )
    return pl.pallas_call(
        body,
        grid_spec=grid_spec,
        out_shape=jax.ShapeDtypeStruct((k, t, b), jnp.float32),
    )(idx, y3)


def kernel(Y_scen):
    s_full, b, t = Y_scen.shape
    k = min(N_SCEN_SEL, s_full)
    if s_full in _PERM_CACHE:
        idx = jnp.asarray(_PERM_CACHE[s_full][:k])
    else:
        idx = jax.random.permutation(jax.random.key(42), s_full)[:k]

    y3 = jnp.transpose(Y_scen, (0, 2, 1))
    p = _sc_one_hot(idx, b, s_full)
    ysel3 = _tc_gather(y3, idx, k, t, b)
    y_sel = ysel3.transpose(0, 2, 1)
    return (y_sel, p, idx)


# speedup vs baseline: 1.1073x; 1.1073x over previous
import functools

import jax
import jax.numpy as jnp
import numpy as np
from jax import lax
from jax.experimental import pallas as pl
from jax.experimental.pallas import tpu as pltpu
from jax.experimental.pallas import tpu_sc as plsc

N_SCEN_SEL = 32

_NC = 2
_NS = 16
_NW = _NC * _NS
_L = 16

_PERM_CACHE = {}
try:
    with jax.default_device(jax.devices("cpu")[0]):
        _PERM_CACHE[512] = np.asarray(
            jax.random.permutation(jax.random.key(42), 512)
        )
except Exception:
    pass

_REP = 4


def _sc_select(y3, idx, idx_np, b, t, s):
    k = idx.shape[0]
    assert k == _NW, "one worker per selected scenario"
    rpw = b // _NW
    assert rpw % _REP == 0 and s % _L == 0
    baked = idx_np is not None

    def body(*refs):
        if baked:
            y_hbm, ysel_hbm, p_hbm, row_v, pat_v, gsem, psem = refs
        else:
            y_hbm, idx_hbm, ysel_hbm, p_hbm, idx_v, row_v, pat_v, gsem, psem = refs
        wid = lax.axis_index("s") * _NC + lax.axis_index("c")
        io = lax.iota(jnp.int32, _L)
        if baked:
            def const_vec(vals):
                v = jnp.full((_L,), int(vals[0]), dtype=jnp.int32)
                for j in range(1, _L):
                    v = jnp.where(io == j, int(vals[j]), v)
                return v

            v_lo = const_vec(idx_np[:_L])
            v_hi = const_vec(idx_np[_L:])
        else:
            pltpu.sync_copy(idx_hbm, idx_v)
            v_lo = idx_v[pl.ds(0, _L)]
            v_hi = idx_v[pl.ds(_L, _L)]
        vsel = jnp.where(wid < _L, v_lo, v_hi)
        g = lax.reduce_max(jnp.where(io == wid % _L, vsel, -1), (0,))
        gcopy = pltpu.async_copy(y_hbm.at[pl.ds(g, 1)], row_v, gsem)

        z16 = jnp.zeros((_L,), jnp.float32)
        cch = s // _L

        def zero_body(kk, _):
            for r in range(_REP):
                for c in range(cch):
                    pat_v[r, kk, pl.ds(c * _L, _L)] = z16
            return 0

        lax.fori_loop(0, k, zero_body, 0)
        ones = jnp.full((_L,), 1.0, dtype=jnp.float32)
        for r in range(_REP):
            rv = jnp.full((_L,), r, dtype=jnp.int32)
            plsc.store_scatter(pat_v, [rv, io, v_lo], ones)
            plsc.store_scatter(pat_v, [rv, io + _L, v_hi], ones)

        pcopies = [
            pltpu.async_copy(
                pat_v, p_hbm.at[pl.ds(wid * rpw + i * _REP, _REP)], psem)
            for i in range(rpw // _REP)
        ]
        gcopy.wait()
        pltpu.sync_copy(row_v, ysel_hbm.at[pl.ds(wid, 1)])
        for c in pcopies:
            c.wait()

    scratch = [
        pltpu.VMEM((1, t, b), jnp.float32),
        pltpu.VMEM((_REP, k, s), jnp.float32),
        pltpu.SemaphoreType.DMA,
        pltpu.SemaphoreType.DMA,
    ]
    if not baked:
        scratch.insert(0, pltpu.VMEM((k,), jnp.int32))

    mesh = plsc.VectorSubcoreMesh(core_axis_name="c", subcore_axis_name="s")
    f = pl.kernel(
        body,
        out_type=(
            jax.ShapeDtypeStruct((k, t, b), jnp.float32),
            jax.ShapeDtypeStruct((b, k, s), jnp.float32),
        ),
        mesh=mesh,
        scratch_types=scratch,
        compiler_params=pltpu.CompilerParams(needs_layout_passes=False),
    )
    return f(y3) if baked else f(y3, idx)


def kernel(Y_scen):
    s_full, b, t = Y_scen.shape
    k = min(N_SCEN_SEL, s_full)
    if s_full in _PERM_CACHE:
        idx_np = _PERM_CACHE[s_full][:k]
        idx = jnp.asarray(idx_np)
    else:
        idx_np = None
        idx = jax.random.permutation(jax.random.key(42), s_full)[:k]

    y3 = jnp.transpose(Y_scen, (0, 2, 1))
    ysel3, p = _sc_select(y3, idx, idx_np, b, t, s_full)
    y_sel = ysel3.transpose(0, 2, 1)
    return (y_sel, p, idx)
